# baseline (speedup 1.0000x reference)
import functools

import jax
import jax.numpy as jnp
from jax import lax
from jax.experimental import pallas as pl
from jax.experimental.pallas import tpu as pltpu
from jax.experimental.pallas import tpu_sc as plsc

N_NODES = 10000
N_EDGES = 320000
D = 128

NC = 2
NS = 16
DH = D // NC

CH = 128
NBUF = 5
N_HBM_SLOTS = 2
NGRP = 32
NCH = NBUF * NGRP
E_TILE = CH * NCH
E_PAD = E_TILE * NS
N_PAD = 10240
ROWS_PER_TILE = N_PAD // NS
DUMMY_DST = N_NODES + 8


def _agg_body(src_hbm, dst_hbm, feat_hbm, zero2_hbm, zero1_hbm, one_hbm,
              acc_out, cnt_out,
              srcb_v, dstb_v, ones_v, rows_v, feat_sh, acc_sh, cnt_sh,
              gsem, ssem, csem, isem, *, with_counts):
    c = lax.axis_index("c")
    s = lax.axis_index("s")
    row0 = s * ROWS_PER_TILE

    pltpu.sync_copy(feat_hbm.at[c, pl.ds(row0, ROWS_PER_TILE)],
                    feat_sh.at[pl.ds(row0, ROWS_PER_TILE)])
    pltpu.sync_copy(zero2_hbm.at[pl.ds(row0, ROWS_PER_TILE)],
                    acc_sh.at[pl.ds(row0, ROWS_PER_TILE)])
    if with_counts:
        @pl.when(c == 0)
        def _():
            pltpu.sync_copy(zero1_hbm.at[pl.ds(row0, ROWS_PER_TILE)],
                            cnt_sh.at[pl.ds(row0, ROWS_PER_TILE)])
            pltpu.sync_copy(one_hbm, ones_v)
    pltpu.sync_copy(src_hbm.at[s, 0], srcb_v.at[pl.ds(0, NBUF)])
    pltpu.sync_copy(dst_hbm.at[s, 0], dstb_v.at[pl.ds(0, NBUF)])
    pltpu.async_copy(src_hbm.at[s, 1], srcb_v.at[pl.ds(NBUF, NBUF)],
                     isem.at[1])
    pltpu.async_copy(dst_hbm.at[s, 1], dstb_v.at[pl.ds(NBUF, NBUF)],
                     isem.at[1])
    plsc.subcore_barrier()

    feat_c = feat_hbm.at[c]

    def gather_src(slot, idx):
        return (feat_c.at[idx] if slot < N_HBM_SLOTS else feat_sh.at[idx])

    for b in range(NBUF):
        pltpu.async_copy(gather_src(b, srcb_v.at[b]), rows_v.at[b],
                         gsem.at[b])

    def half(g2, par):
        pbase = par * NBUF
        qbase = (1 - par) * NBUF
        g = 2 * g2 + par
        scat = []
        for b in range(NBUF):
            pltpu.make_async_copy(gather_src(b, srcb_v.at[0]),
                                  rows_v.at[b], gsem.at[b]).wait()
            scat.append(pltpu.async_copy(
                rows_v.at[b], acc_sh.at[dstb_v.at[pbase + b]], ssem.at[b],
                add=True))
            if with_counts:
                @pl.when(c == 0)
                def _():
                    pltpu.async_copy(ones_v, cnt_sh.at[dstb_v.at[pbase + b]],
                                     csem, add=True)
        for b in range(NBUF):
            scat[b].wait()
        if with_counts:
            @pl.when(c == 0)
            def _():
                pltpu.make_async_copy(src_hbm.at[s, 0],
                                      srcb_v.at[pl.ds(0, NBUF)], csem).wait()

        @pl.when(g + 1 < NGRP)
        def _():
            pltpu.make_async_copy(src_hbm.at[s, 0],
                                  srcb_v.at[pl.ds(qbase, NBUF)],
                                  isem.at[1 - par]).wait()
            pltpu.make_async_copy(dst_hbm.at[s, 0],
                                  dstb_v.at[pl.ds(qbase, NBUF)],
                                  isem.at[1 - par]).wait()
            for b in range(NBUF):
                pltpu.async_copy(gather_src(b, srcb_v.at[qbase + b]),
                                 rows_v.at[b], gsem.at[b])

        @pl.when(g + 2 < NGRP)
        def _():
            pltpu.async_copy(src_hbm.at[s, g + 2],
                             srcb_v.at[pl.ds(pbase, NBUF)], isem.at[par])
            pltpu.async_copy(dst_hbm.at[s, g + 2],
                             dstb_v.at[pl.ds(pbase, NBUF)], isem.at[par])

    def super_group(g2, carry):
        half(g2, 0)
        half(g2, 1)
        return carry

    lax.fori_loop(0, NGRP // 2, super_group, 0)
    plsc.subcore_barrier()

    pltpu.sync_copy(acc_sh.at[pl.ds(row0, ROWS_PER_TILE)],
                    acc_out.at[c, pl.ds(row0, ROWS_PER_TILE)])
    if with_counts:
        @pl.when(c == 0)
        def _():
            pltpu.sync_copy(cnt_sh.at[pl.ds(row0, ROWS_PER_TILE)],
                            cnt_out.at[pl.ds(row0, ROWS_PER_TILE)])


def _aggregate(src, dst, feat, zero2, zero1, one, with_counts):
    mesh = plsc.VectorSubcoreMesh(core_axis_name="c", subcore_axis_name="s")
    f = pl.kernel(
        functools.partial(_agg_body, with_counts=with_counts),
        out_type=[
            jax.ShapeDtypeStruct((NC, N_PAD, DH), jnp.float32),
            jax.ShapeDtypeStruct((N_PAD,), jnp.float32),
        ],
        mesh=mesh,
        scratch_types=[
            pltpu.VMEM((2 * NBUF, CH), jnp.int32),
            pltpu.VMEM((2 * NBUF, CH), jnp.int32),
            pltpu.VMEM((CH,), jnp.float32),
            pltpu.VMEM((NBUF, CH, DH), jnp.float32),
            pltpu.VMEM_SHARED((N_PAD, DH), jnp.float32),
            pltpu.VMEM_SHARED((N_PAD, DH), jnp.float32),
            pltpu.VMEM_SHARED((N_PAD,), jnp.float32),
            pltpu.SemaphoreType.DMA((NBUF,)),
            pltpu.SemaphoreType.DMA((NBUF,)),
            pltpu.SemaphoreType.DMA,
            pltpu.SemaphoreType.DMA((2,)),
        ],
        compiler_params=pltpu.CompilerParams(use_tc_tiling_on_sc=False),
    )
    return f(src, dst, feat, zero2, zero1, one)


def _dense1_body(a0_ref, a1_ref, cnt_ref, x0_ref, x1_ref, wl_ref, wr_ref,
                 b_ref, o_ref):
    cnt = jnp.maximum(cnt_ref[...], 1.0)
    mean = jnp.concatenate([a0_ref[0], a1_ref[0]], axis=1) / cnt
    x = jnp.concatenate([x0_ref[0], x1_ref[0]], axis=1)
    out = (jnp.dot(mean, wl_ref[0], preferred_element_type=jnp.float32)
           + b_ref[0]
           + jnp.dot(x, wr_ref[0], preferred_element_type=jnp.float32))
    o_ref[0] = jnp.maximum(out, 0.0)


def _dense2_body(a0_ref, a1_ref, cnt_ref, x0_ref, x1_ref, wl_ref, wr_ref,
                 b_ref, o_ref):
    cnt = jnp.maximum(cnt_ref[...], 1.0)
    mean = jnp.concatenate([a0_ref[0], a1_ref[0]], axis=1) / cnt
    x = jnp.concatenate([x0_ref[0], x1_ref[0]], axis=1)
    out = (jnp.dot(mean, wl_ref[...], preferred_element_type=jnp.float32)
           + b_ref[...]
           + jnp.dot(x, wr_ref[...], preferred_element_type=jnp.float32))
    out = jnp.where(jnp.isnan(out), jnp.float32(0.0), out)
    out = jnp.where(out == jnp.inf, jnp.float32(10000.0), out)
    out = jnp.where(out == -jnp.inf, jnp.float32(-10000.0), out)
    o_ref[...] = out


_BLK1 = 640
_BLK2 = 400


def _dense1(acc, cnt, xs, wl_s, wr_s, b_s):
    return pl.pallas_call(
        _dense1_body,
        grid=(NC, N_PAD // _BLK1),
        in_specs=[
            pl.BlockSpec((1, _BLK1, DH), lambda j, i: (0, i, 0)),
            pl.BlockSpec((1, _BLK1, DH), lambda j, i: (1, i, 0)),
            pl.BlockSpec((_BLK1, 1), lambda j, i: (i, 0)),
            pl.BlockSpec((1, _BLK1, DH), lambda j, i: (0, i, 0)),
            pl.BlockSpec((1, _BLK1, DH), lambda j, i: (1, i, 0)),
            pl.BlockSpec((1, D, DH), lambda j, i: (j, 0, 0)),
            pl.BlockSpec((1, D, DH), lambda j, i: (j, 0, 0)),
            pl.BlockSpec((1, 1, DH), lambda j, i: (j, 0, 0)),
        ],
        out_specs=pl.BlockSpec((1, _BLK1, DH), lambda j, i: (j, i, 0)),
        out_shape=jax.ShapeDtypeStruct((NC, N_PAD, DH), jnp.float32),
    )(acc, acc, cnt, xs, xs, wl_s, wr_s, b_s)


def _dense2(acc, cnt, xs, wl_t, wr_t, b):
    return pl.pallas_call(
        _dense2_body,
        grid=(N_NODES // _BLK2,),
        in_specs=[
            pl.BlockSpec((1, _BLK2, DH), lambda i: (0, i, 0)),
            pl.BlockSpec((1, _BLK2, DH), lambda i: (1, i, 0)),
            pl.BlockSpec((_BLK2, 1), lambda i: (i, 0)),
            pl.BlockSpec((1, _BLK2, DH), lambda i: (0, i, 0)),
            pl.BlockSpec((1, _BLK2, DH), lambda i: (1, i, 0)),
            pl.BlockSpec((D, D), lambda i: (0, 0)),
            pl.BlockSpec((D, D), lambda i: (0, 0)),
            pl.BlockSpec((1, D), lambda i: (0, 0)),
        ],
        out_specs=pl.BlockSpec((_BLK2, D), lambda i: (i, 0)),
        out_shape=jax.ShapeDtypeStruct((N_NODES, D), jnp.float32),
    )(acc, acc, cnt, xs, xs, wl_t, wr_t, b)


def _split_cols_pad(x):
    xp = jnp.zeros((NC, N_PAD, DH), jnp.float32)
    return xp.at[:, :x.shape[0]].set(jnp.stack([x[:, :DH], x[:, DH:]]))


def _split_cols(w):
    return jnp.stack([w[:, :DH], w[:, DH:]])


def kernel(x, edge_index, W1_l, b1_l, W1_r, W2_l, b2_l, W2_r):
    src = edge_index[0].astype(jnp.int32)
    dst = edge_index[1].astype(jnp.int32)
    pad = E_PAD - N_EDGES
    src = jnp.concatenate([src, jnp.zeros((pad,), jnp.int32)])
    dst = jnp.concatenate([dst, jnp.full((pad,), DUMMY_DST, jnp.int32)])
    src = src.reshape(NS, NGRP, NBUF, CH)
    dst = dst.reshape(NS, NGRP, NBUF, CH)

    zero2 = jnp.zeros((N_PAD, DH), jnp.float32)
    zero1 = jnp.zeros((N_PAD,), jnp.float32)
    one = jnp.ones((CH,), jnp.float32)

    xs = _split_cols_pad(x)

    acc, cnt = _aggregate(src, dst, xs, zero2, zero1, one, with_counts=True)
    cnt2 = jnp.maximum(cnt, 1.0)[:, None]

    hs = _dense1(acc, cnt2, xs, _split_cols(W1_l.T), _split_cols(W1_r.T),
                 _split_cols(b1_l[None, :]))

    acc2, _ = _aggregate(src, dst, hs, zero2, zero1, one, with_counts=False)
    out = _dense2(acc2, cnt2, hs, W2_l.T, W2_r.T, b2_l[None, :])
    return out


# speedup vs baseline: 1.0737x; 1.0737x over previous
import functools

import jax
import jax.numpy as jnp
from jax import lax
from jax.experimental import pallas as pl
from jax.experimental.pallas import tpu as pltpu
from jax.experimental.pallas import tpu_sc as plsc

N_NODES = 10000
N_EDGES = 320000
D = 128

NC = 2
NS = 16
DH = D // NC

CH = 128
NBUF = 5
N_HBM_SLOTS = 0
NGRP = 32
NCH = NBUF * NGRP
E_TILE = CH * NCH
E_PAD = E_TILE * NS
N_PAD = 10240
ROWS_PER_TILE = N_PAD // NS
DUMMY_DST = N_NODES + 8


def _agg_body(src_hbm, dst_hbm, feat_hbm, zero2_hbm, zero1_hbm, one_hbm,
              acc_out, cnt_out,
              srcb_v, dstb_v, ones_v, rows_v, feat_sh, acc_sh, cnt_sh,
              gsem, ssem, csem, isem, *, with_counts):
    c = lax.axis_index("c")
    s = lax.axis_index("s")
    row0 = s * ROWS_PER_TILE

    pltpu.sync_copy(feat_hbm.at[c, pl.ds(row0, ROWS_PER_TILE)],
                    feat_sh.at[pl.ds(row0, ROWS_PER_TILE)])
    pltpu.sync_copy(zero2_hbm.at[pl.ds(row0, ROWS_PER_TILE)],
                    acc_sh.at[pl.ds(row0, ROWS_PER_TILE)])
    if with_counts:
        @pl.when(c == 0)
        def _():
            pltpu.sync_copy(zero1_hbm.at[pl.ds(row0, ROWS_PER_TILE)],
                            cnt_sh.at[pl.ds(row0, ROWS_PER_TILE)])
            pltpu.sync_copy(one_hbm, ones_v)
    pltpu.sync_copy(src_hbm.at[s, 0], srcb_v.at[pl.ds(0, NBUF)])
    pltpu.sync_copy(dst_hbm.at[s, 0], dstb_v.at[pl.ds(0, NBUF)])
    pltpu.async_copy(src_hbm.at[s, 1], srcb_v.at[pl.ds(NBUF, NBUF)],
                     isem.at[1])
    pltpu.async_copy(dst_hbm.at[s, 1], dstb_v.at[pl.ds(NBUF, NBUF)],
                     isem.at[1])
    plsc.subcore_barrier()

    feat_c = feat_hbm.at[c]

    def gather_src(slot, idx):
        return (feat_c.at[idx] if slot < N_HBM_SLOTS else feat_sh.at[idx])

    for b in range(NBUF):
        pltpu.async_copy(gather_src(b, srcb_v.at[b]), rows_v.at[b],
                         gsem.at[b])

    def half(g2, par):
        pbase = par * NBUF
        qbase = (1 - par) * NBUF
        g = 2 * g2 + par
        scat = []
        for b in range(NBUF):
            pltpu.make_async_copy(gather_src(b, srcb_v.at[0]),
                                  rows_v.at[b], gsem.at[b]).wait()
            scat.append(pltpu.async_copy(
                rows_v.at[b], acc_sh.at[dstb_v.at[pbase + b]], ssem.at[b],
                add=True))
            if with_counts:
                @pl.when(c == 0)
                def _():
                    pltpu.async_copy(ones_v, cnt_sh.at[dstb_v.at[pbase + b]],
                                     csem, add=True)
        for b in range(NBUF):
            scat[b].wait()
        if with_counts:
            @pl.when(c == 0)
            def _():
                pltpu.make_async_copy(src_hbm.at[s, 0],
                                      srcb_v.at[pl.ds(0, NBUF)], csem).wait()

        @pl.when(g + 1 < NGRP)
        def _():
            pltpu.make_async_copy(src_hbm.at[s, 0],
                                  srcb_v.at[pl.ds(qbase, NBUF)],
                                  isem.at[1 - par]).wait()
            pltpu.make_async_copy(dst_hbm.at[s, 0],
                                  dstb_v.at[pl.ds(qbase, NBUF)],
                                  isem.at[1 - par]).wait()
            for b in range(NBUF):
                pltpu.async_copy(gather_src(b, srcb_v.at[qbase + b]),
                                 rows_v.at[b], gsem.at[b])

        @pl.when(g + 2 < NGRP)
        def _():
            pltpu.async_copy(src_hbm.at[s, g + 2],
                             srcb_v.at[pl.ds(pbase, NBUF)], isem.at[par])
            pltpu.async_copy(dst_hbm.at[s, g + 2],
                             dstb_v.at[pl.ds(pbase, NBUF)], isem.at[par])

    def super_group(g2, carry):
        half(g2, 0)
        half(g2, 1)
        return carry

    lax.fori_loop(0, NGRP // 2, super_group, 0)
    plsc.subcore_barrier()

    pltpu.sync_copy(acc_sh.at[pl.ds(row0, ROWS_PER_TILE)],
                    acc_out.at[c, pl.ds(row0, ROWS_PER_TILE)])
    if with_counts:
        @pl.when(c == 0)
        def _():
            pltpu.sync_copy(cnt_sh.at[pl.ds(row0, ROWS_PER_TILE)],
                            cnt_out.at[pl.ds(row0, ROWS_PER_TILE)])


def _aggregate(src, dst, feat, zero2, zero1, one, with_counts):
    mesh = plsc.VectorSubcoreMesh(core_axis_name="c", subcore_axis_name="s")
    f = pl.kernel(
        functools.partial(_agg_body, with_counts=with_counts),
        out_type=[
            jax.ShapeDtypeStruct((NC, N_PAD, DH), jnp.float32),
            jax.ShapeDtypeStruct((N_PAD,), jnp.float32),
        ],
        mesh=mesh,
        scratch_types=[
            pltpu.VMEM((2 * NBUF, CH), jnp.int32),
            pltpu.VMEM((2 * NBUF, CH), jnp.int32),
            pltpu.VMEM((CH,), jnp.float32),
            pltpu.VMEM((NBUF, CH, DH), jnp.float32),
            pltpu.VMEM_SHARED((N_PAD, DH), jnp.float32),
            pltpu.VMEM_SHARED((N_PAD, DH), jnp.float32),
            pltpu.VMEM_SHARED((N_PAD,), jnp.float32),
            pltpu.SemaphoreType.DMA((NBUF,)),
            pltpu.SemaphoreType.DMA((NBUF,)),
            pltpu.SemaphoreType.DMA,
            pltpu.SemaphoreType.DMA((2,)),
        ],
        compiler_params=pltpu.CompilerParams(use_tc_tiling_on_sc=False),
    )
    return f(src, dst, feat, zero2, zero1, one)


def _dense1_body(a0_ref, a1_ref, cnt_ref, x0_ref, x1_ref, wl_ref, wr_ref,
                 b_ref, o_ref):
    cnt = jnp.maximum(cnt_ref[...], 1.0)
    mean = jnp.concatenate([a0_ref[0], a1_ref[0]], axis=1) / cnt
    x = jnp.concatenate([x0_ref[0], x1_ref[0]], axis=1)
    out = (jnp.dot(mean, wl_ref[0], preferred_element_type=jnp.float32)
           + b_ref[0]
           + jnp.dot(x, wr_ref[0], preferred_element_type=jnp.float32))
    o_ref[0] = jnp.maximum(out, 0.0)


def _dense2_body(a0_ref, a1_ref, cnt_ref, x0_ref, x1_ref, wl_ref, wr_ref,
                 b_ref, o_ref):
    cnt = jnp.maximum(cnt_ref[...], 1.0)
    mean = jnp.concatenate([a0_ref[0], a1_ref[0]], axis=1) / cnt
    x = jnp.concatenate([x0_ref[0], x1_ref[0]], axis=1)
    out = (jnp.dot(mean, wl_ref[...], preferred_element_type=jnp.float32)
           + b_ref[...]
           + jnp.dot(x, wr_ref[...], preferred_element_type=jnp.float32))
    out = jnp.where(jnp.isnan(out), jnp.float32(0.0), out)
    out = jnp.where(out == jnp.inf, jnp.float32(10000.0), out)
    out = jnp.where(out == -jnp.inf, jnp.float32(-10000.0), out)
    o_ref[...] = out


_BLK1 = 640
_BLK2 = 400


def _dense1(acc, cnt, xs, wl_s, wr_s, b_s):
    return pl.pallas_call(
        _dense1_body,
        grid=(NC, N_PAD // _BLK1),
        in_specs=[
            pl.BlockSpec((1, _BLK1, DH), lambda j, i: (0, i, 0)),
            pl.BlockSpec((1, _BLK1, DH), lambda j, i: (1, i, 0)),
            pl.BlockSpec((_BLK1, 1), lambda j, i: (i, 0)),
            pl.BlockSpec((1, _BLK1, DH), lambda j, i: (0, i, 0)),
            pl.BlockSpec((1, _BLK1, DH), lambda j, i: (1, i, 0)),
            pl.BlockSpec((1, D, DH), lambda j, i: (j, 0, 0)),
            pl.BlockSpec((1, D, DH), lambda j, i: (j, 0, 0)),
            pl.BlockSpec((1, 1, DH), lambda j, i: (j, 0, 0)),
        ],
        out_specs=pl.BlockSpec((1, _BLK1, DH), lambda j, i: (j, i, 0)),
        out_shape=jax.ShapeDtypeStruct((NC, N_PAD, DH), jnp.float32),
    )(acc, acc, cnt, xs, xs, wl_s, wr_s, b_s)


def _dense2(acc, cnt, xs, wl_t, wr_t, b):
    return pl.pallas_call(
        _dense2_body,
        grid=(N_NODES // _BLK2,),
        in_specs=[
            pl.BlockSpec((1, _BLK2, DH), lambda i: (0, i, 0)),
            pl.BlockSpec((1, _BLK2, DH), lambda i: (1, i, 0)),
            pl.BlockSpec((_BLK2, 1), lambda i: (i, 0)),
            pl.BlockSpec((1, _BLK2, DH), lambda i: (0, i, 0)),
            pl.BlockSpec((1, _BLK2, DH), lambda i: (1, i, 0)),
            pl.BlockSpec((D, D), lambda i: (0, 0)),
            pl.BlockSpec((D, D), lambda i: (0, 0)),
            pl.BlockSpec((1, D), lambda i: (0, 0)),
        ],
        out_specs=pl.BlockSpec((_BLK2, D), lambda i: (i, 0)),
        out_shape=jax.ShapeDtypeStruct((N_NODES, D), jnp.float32),
    )(acc, acc, cnt, xs, xs, wl_t, wr_t, b)


def _split_cols_pad(x):
    xp = jnp.zeros((NC, N_PAD, DH), jnp.float32)
    return xp.at[:, :x.shape[0]].set(jnp.stack([x[:, :DH], x[:, DH:]]))


def _split_cols(w):
    return jnp.stack([w[:, :DH], w[:, DH:]])


def kernel(x, edge_index, W1_l, b1_l, W1_r, W2_l, b2_l, W2_r):
    src = edge_index[0].astype(jnp.int32)
    dst = edge_index[1].astype(jnp.int32)
    pad = E_PAD - N_EDGES
    src = jnp.concatenate([src, jnp.zeros((pad,), jnp.int32)])
    dst = jnp.concatenate([dst, jnp.full((pad,), DUMMY_DST, jnp.int32)])
    src = src.reshape(NS, NGRP, NBUF, CH)
    dst = dst.reshape(NS, NGRP, NBUF, CH)

    zero2 = jnp.zeros((N_PAD, DH), jnp.float32)
    zero1 = jnp.zeros((N_PAD,), jnp.float32)
    one = jnp.ones((CH,), jnp.float32)

    xs = _split_cols_pad(x)

    acc, cnt = _aggregate(src, dst, xs, zero2, zero1, one, with_counts=True)
    cnt2 = jnp.maximum(cnt, 1.0)[:, None]

    hs = _dense1(acc, cnt2, xs, _split_cols(W1_l.T), _split_cols(W1_r.T),
                 _split_cols(b1_l[None, :]))

    acc2, _ = _aggregate(src, dst, hs, zero2, zero1, one, with_counts=False)
    out = _dense2(acc2, cnt2, hs, W2_l.T, W2_r.T, b2_l[None, :])
    return out


# speedup vs baseline: 1.1291x; 1.0517x over previous
import functools

import jax
import jax.numpy as jnp
from jax import lax
from jax.experimental import pallas as pl
from jax.experimental.pallas import tpu as pltpu
from jax.experimental.pallas import tpu_sc as plsc

N_NODES = 10000
N_EDGES = 320000
D = 128

NC = 2
NS = 16
DH = D // NC

CH = 128
NBUF = 5
N_HBM_SLOTS = 0
NGRP = 32
NCH = NBUF * NGRP
E_TILE = CH * NCH
E_PAD = E_TILE * NS
N_PAD = 10240
ROWS_PER_TILE = N_PAD // NS
DUMMY_DST = N_NODES + 8


def _agg_body(src_hbm, dst_hbm, feat_hbm, zero2_hbm, zero1_hbm, one_hbm,
              acc_out, cnt_out,
              srcb_v, dstb_v, ones_v, rows_v, feat_sh, acc_sh, cnt_sh,
              gsem, ssem, csem, isem, *, with_counts):
    c = lax.axis_index("c")
    s = lax.axis_index("s")
    row0 = s * ROWS_PER_TILE

    pltpu.sync_copy(feat_hbm.at[c, pl.ds(row0, ROWS_PER_TILE)],
                    feat_sh.at[pl.ds(row0, ROWS_PER_TILE)])
    pltpu.sync_copy(zero2_hbm.at[pl.ds(row0, ROWS_PER_TILE)],
                    acc_sh.at[pl.ds(row0, ROWS_PER_TILE)])
    if with_counts:
        @pl.when(c == 0)
        def _():
            pltpu.sync_copy(zero1_hbm.at[pl.ds(row0, ROWS_PER_TILE)],
                            cnt_sh.at[pl.ds(row0, ROWS_PER_TILE)])
            pltpu.sync_copy(one_hbm, ones_v)
    pltpu.sync_copy(src_hbm.at[s, 0], srcb_v.at[pl.ds(0, NBUF)])
    pltpu.sync_copy(dst_hbm.at[s, 0], dstb_v.at[pl.ds(0, NBUF)])
    pltpu.async_copy(src_hbm.at[s, 1], srcb_v.at[pl.ds(NBUF, NBUF)],
                     isem.at[1])
    pltpu.async_copy(dst_hbm.at[s, 1], dstb_v.at[pl.ds(NBUF, NBUF)],
                     isem.at[1])
    plsc.subcore_barrier()

    feat_c = feat_hbm.at[c]

    def gather_src(slot, idx):
        return (feat_c.at[idx] if slot < N_HBM_SLOTS else feat_sh.at[idx])

    for b in range(NBUF):
        pltpu.async_copy(gather_src(b, srcb_v.at[b]), rows_v.at[b],
                         gsem.at[b])

    def half(g2, par):
        pbase = par * NBUF
        qbase = (1 - par) * NBUF
        g = 2 * g2 + par
        scat = []
        for b in range(NBUF):
            pltpu.make_async_copy(gather_src(b, srcb_v.at[0]),
                                  rows_v.at[b], gsem.at[b]).wait()
            scat.append(pltpu.async_copy(
                rows_v.at[b], acc_sh.at[dstb_v.at[pbase + b]], ssem.at[b],
                add=True))
            if with_counts:
                @pl.when(c == 0)
                def _():
                    pltpu.async_copy(ones_v, cnt_sh.at[dstb_v.at[pbase + b]],
                                     csem, add=True)
        for b in range(NBUF):
            scat[b].wait()
        if with_counts:
            @pl.when(c == 0)
            def _():
                pltpu.make_async_copy(src_hbm.at[s, 0],
                                      srcb_v.at[pl.ds(0, NBUF)], csem).wait()

        @pl.when(g + 1 < NGRP)
        def _():
            pltpu.make_async_copy(src_hbm.at[s, 0],
                                  srcb_v.at[pl.ds(qbase, NBUF)],
                                  isem.at[1 - par]).wait()
            pltpu.make_async_copy(dst_hbm.at[s, 0],
                                  dstb_v.at[pl.ds(qbase, NBUF)],
                                  isem.at[1 - par]).wait()
            for b in range(NBUF):
                pltpu.async_copy(gather_src(b, srcb_v.at[qbase + b]),
                                 rows_v.at[b], gsem.at[b])

        @pl.when(g + 2 < NGRP)
        def _():
            pltpu.async_copy(src_hbm.at[s, g + 2],
                             srcb_v.at[pl.ds(pbase, NBUF)], isem.at[par])
            pltpu.async_copy(dst_hbm.at[s, g + 2],
                             dstb_v.at[pl.ds(pbase, NBUF)], isem.at[par])

    def super_group(g2, carry):
        half(g2, 0)
        half(g2, 1)
        return carry

    lax.fori_loop(0, NGRP // 2, super_group, 0)
    plsc.subcore_barrier()

    pltpu.sync_copy(acc_sh.at[pl.ds(row0, ROWS_PER_TILE)],
                    acc_out.at[c, pl.ds(row0, ROWS_PER_TILE)])
    if with_counts:
        @pl.when(c == 0)
        def _():
            pltpu.sync_copy(cnt_sh.at[pl.ds(row0, ROWS_PER_TILE)],
                            cnt_out.at[pl.ds(row0, ROWS_PER_TILE)])


def _aggregate(src, dst, feat, zero2, zero1, one, with_counts):
    mesh = plsc.VectorSubcoreMesh(core_axis_name="c", subcore_axis_name="s")
    f = pl.kernel(
        functools.partial(_agg_body, with_counts=with_counts),
        out_type=[
            jax.ShapeDtypeStruct((NC, N_PAD, DH), jnp.float32),
            jax.ShapeDtypeStruct((N_PAD,), jnp.float32),
        ],
        mesh=mesh,
        scratch_types=[
            pltpu.VMEM((2 * NBUF, CH), jnp.int32),
            pltpu.VMEM((2 * NBUF, CH), jnp.int32),
            pltpu.VMEM((CH,), jnp.float32),
            pltpu.VMEM((NBUF, CH, DH), jnp.float32),
            pltpu.VMEM_SHARED((N_PAD, DH), jnp.float32),
            pltpu.VMEM_SHARED((N_PAD, DH), jnp.float32),
            pltpu.VMEM_SHARED((N_PAD,), jnp.float32),
            pltpu.SemaphoreType.DMA((NBUF,)),
            pltpu.SemaphoreType.DMA((NBUF,)),
            pltpu.SemaphoreType.DMA,
            pltpu.SemaphoreType.DMA((2,)),
        ],
        compiler_params=pltpu.CompilerParams(use_tc_tiling_on_sc=False),
    )
    return f(src, dst, feat, zero2, zero1, one)


def _dense1_body(a_ref, cnt_ref, x_ref, wl_ref, wr_ref, b_ref, o_ref):
    cnt = jnp.maximum(cnt_ref[...], 1.0)
    mean = jnp.concatenate([a_ref[0], a_ref[1]], axis=1) / cnt
    x = jnp.concatenate([x_ref[0], x_ref[1]], axis=1)
    out = (jnp.dot(mean, wl_ref[...], preferred_element_type=jnp.float32)
           + b_ref[...]
           + jnp.dot(x, wr_ref[...], preferred_element_type=jnp.float32))
    out = jnp.maximum(out, 0.0)
    o_ref[0] = out[:, :DH]
    o_ref[1] = out[:, DH:]


def _dense2_body(a0_ref, a1_ref, cnt_ref, x0_ref, x1_ref, wl_ref, wr_ref,
                 b_ref, o_ref):
    cnt = jnp.maximum(cnt_ref[...], 1.0)
    mean = jnp.concatenate([a0_ref[0], a1_ref[0]], axis=1) / cnt
    x = jnp.concatenate([x0_ref[0], x1_ref[0]], axis=1)
    out = (jnp.dot(mean, wl_ref[...], preferred_element_type=jnp.float32)
           + b_ref[...]
           + jnp.dot(x, wr_ref[...], preferred_element_type=jnp.float32))
    out = jnp.where(jnp.isnan(out), jnp.float32(0.0), out)
    out = jnp.where(out == jnp.inf, jnp.float32(10000.0), out)
    out = jnp.where(out == -jnp.inf, jnp.float32(-10000.0), out)
    o_ref[...] = out


_BLK1 = 1280
_BLK2 = 1000


def _dense1(acc, cnt, xs, wl_t, wr_t, b):
    return pl.pallas_call(
        _dense1_body,
        grid=(N_PAD // _BLK1,),
        in_specs=[
            pl.BlockSpec((NC, _BLK1, DH), lambda i: (0, i, 0)),
            pl.BlockSpec((_BLK1, 1), lambda i: (i, 0)),
            pl.BlockSpec((NC, _BLK1, DH), lambda i: (0, i, 0)),
            pl.BlockSpec((D, D), lambda i: (0, 0)),
            pl.BlockSpec((D, D), lambda i: (0, 0)),
            pl.BlockSpec((1, D), lambda i: (0, 0)),
        ],
        out_specs=pl.BlockSpec((NC, _BLK1, DH), lambda i: (0, i, 0)),
        out_shape=jax.ShapeDtypeStruct((NC, N_PAD, DH), jnp.float32),
    )(acc, cnt, xs, wl_t, wr_t, b)


def _dense2(acc, cnt, xs, wl_t, wr_t, b):
    return pl.pallas_call(
        _dense2_body,
        grid=(N_NODES // _BLK2,),
        in_specs=[
            pl.BlockSpec((1, _BLK2, DH), lambda i: (0, i, 0)),
            pl.BlockSpec((1, _BLK2, DH), lambda i: (1, i, 0)),
            pl.BlockSpec((_BLK2, 1), lambda i: (i, 0)),
            pl.BlockSpec((1, _BLK2, DH), lambda i: (0, i, 0)),
            pl.BlockSpec((1, _BLK2, DH), lambda i: (1, i, 0)),
            pl.BlockSpec((D, D), lambda i: (0, 0)),
            pl.BlockSpec((D, D), lambda i: (0, 0)),
            pl.BlockSpec((1, D), lambda i: (0, 0)),
        ],
        out_specs=pl.BlockSpec((_BLK2, D), lambda i: (i, 0)),
        out_shape=jax.ShapeDtypeStruct((N_NODES, D), jnp.float32),
    )(acc, acc, cnt, xs, xs, wl_t, wr_t, b)


def _split_cols_pad(x):
    xp = jnp.zeros((NC, N_PAD, DH), jnp.float32)
    return xp.at[:, :x.shape[0]].set(jnp.stack([x[:, :DH], x[:, DH:]]))


def kernel(x, edge_index, W1_l, b1_l, W1_r, W2_l, b2_l, W2_r):
    src = edge_index[0].astype(jnp.int32)
    dst = edge_index[1].astype(jnp.int32)
    pad = E_PAD - N_EDGES
    src = jnp.concatenate([src, jnp.zeros((pad,), jnp.int32)])
    dst = jnp.concatenate([dst, jnp.full((pad,), DUMMY_DST, jnp.int32)])
    src = src.reshape(NS, NGRP, NBUF, CH)
    dst = dst.reshape(NS, NGRP, NBUF, CH)

    zero2 = jnp.zeros((N_PAD, DH), jnp.float32)
    zero1 = jnp.zeros((N_PAD,), jnp.float32)
    one = jnp.ones((CH,), jnp.float32)

    xs = _split_cols_pad(x)

    acc, cnt = _aggregate(src, dst, xs, zero2, zero1, one, with_counts=True)
    cnt2 = jnp.maximum(cnt, 1.0)[:, None]

    hs = _dense1(acc, cnt2, xs, W1_l.T, W1_r.T, b1_l[None, :])

    acc2, _ = _aggregate(src, dst, hs, zero2, zero1, one, with_counts=False)
    out = _dense2(acc2, cnt2, hs, W2_l.T, W2_r.T, b2_l[None, :])
    return out


# speedup vs baseline: 1.1305x; 1.0012x over previous
import functools

import jax
import jax.numpy as jnp
from jax import lax
from jax.experimental import pallas as pl
from jax.experimental.pallas import tpu as pltpu
from jax.experimental.pallas import tpu_sc as plsc

N_NODES = 10000
N_EDGES = 320000
D = 128

NC = 2
NS = 16
DH = D // NC

CH = 128
NBUF = 5
N_HBM_SLOTS = 0
NGRP = 32
NCH = NBUF * NGRP
E_TILE = CH * NCH
E_PAD = E_TILE * NS
N_PAD = 10240
ROWS_PER_TILE = N_PAD // NS
DUMMY_DST = N_NODES + 8


def _agg_body(src_hbm, dst_hbm, feat_hbm, zero2_hbm, zero1_hbm, one_hbm,
              acc_out, cnt_out,
              srcb_v, dstb_v, ones_v, rows_v, feat_sh, acc_sh, cnt_sh,
              gsem, ssem, csem, isem, *, with_counts):
    c = lax.axis_index("c")
    s = lax.axis_index("s")
    row0 = s * ROWS_PER_TILE

    pltpu.sync_copy(feat_hbm.at[c, pl.ds(row0, ROWS_PER_TILE)],
                    feat_sh.at[pl.ds(row0, ROWS_PER_TILE)])
    pltpu.sync_copy(zero2_hbm.at[pl.ds(row0, ROWS_PER_TILE)],
                    acc_sh.at[pl.ds(row0, ROWS_PER_TILE)])
    if with_counts:
        pltpu.sync_copy(zero1_hbm.at[pl.ds(row0, ROWS_PER_TILE)],
                        cnt_sh.at[pl.ds(row0, ROWS_PER_TILE)])
        pltpu.sync_copy(one_hbm, ones_v)
    pltpu.sync_copy(src_hbm.at[s, 0], srcb_v.at[pl.ds(0, NBUF)])
    pltpu.sync_copy(dst_hbm.at[s, 0], dstb_v.at[pl.ds(0, NBUF)])
    pltpu.async_copy(src_hbm.at[s, 1], srcb_v.at[pl.ds(NBUF, NBUF)],
                     isem.at[1])
    pltpu.async_copy(dst_hbm.at[s, 1], dstb_v.at[pl.ds(NBUF, NBUF)],
                     isem.at[1])
    plsc.subcore_barrier()

    feat_c = feat_hbm.at[c]

    def gather_src(slot, idx):
        return (feat_c.at[idx] if slot < N_HBM_SLOTS else feat_sh.at[idx])

    for b in range(NBUF):
        pltpu.async_copy(gather_src(b, srcb_v.at[b]), rows_v.at[b],
                         gsem.at[b])

    def half(g2, par):
        pbase = par * NBUF
        qbase = (1 - par) * NBUF
        g = 2 * g2 + par
        cnt_here = jnp.logical_or(
            jnp.logical_and(c == 0, g < NGRP // 2),
            jnp.logical_and(c == 1, g >= NGRP // 2))
        scat = []
        for b in range(NBUF):
            pltpu.make_async_copy(gather_src(b, srcb_v.at[0]),
                                  rows_v.at[b], gsem.at[b]).wait()
            scat.append(pltpu.async_copy(
                rows_v.at[b], acc_sh.at[dstb_v.at[pbase + b]], ssem.at[b],
                add=True))
            if with_counts:
                @pl.when(cnt_here)
                def _():
                    pltpu.async_copy(ones_v, cnt_sh.at[dstb_v.at[pbase + b]],
                                     csem, add=True)
        for b in range(NBUF):
            scat[b].wait()
        if with_counts:
            @pl.when(cnt_here)
            def _():
                pltpu.make_async_copy(src_hbm.at[s, 0],
                                      srcb_v.at[pl.ds(0, NBUF)], csem).wait()

        @pl.when(g + 1 < NGRP)
        def _():
            pltpu.make_async_copy(src_hbm.at[s, 0],
                                  srcb_v.at[pl.ds(qbase, NBUF)],
                                  isem.at[1 - par]).wait()
            pltpu.make_async_copy(dst_hbm.at[s, 0],
                                  dstb_v.at[pl.ds(qbase, NBUF)],
                                  isem.at[1 - par]).wait()
            for b in range(NBUF):
                pltpu.async_copy(gather_src(b, srcb_v.at[qbase + b]),
                                 rows_v.at[b], gsem.at[b])

        @pl.when(g + 2 < NGRP)
        def _():
            pltpu.async_copy(src_hbm.at[s, g + 2],
                             srcb_v.at[pl.ds(pbase, NBUF)], isem.at[par])
            pltpu.async_copy(dst_hbm.at[s, g + 2],
                             dstb_v.at[pl.ds(pbase, NBUF)], isem.at[par])

    def super_group(g2, carry):
        half(g2, 0)
        half(g2, 1)
        return carry

    lax.fori_loop(0, NGRP // 2, super_group, 0)
    plsc.subcore_barrier()

    pltpu.sync_copy(acc_sh.at[pl.ds(row0, ROWS_PER_TILE)],
                    acc_out.at[c, pl.ds(row0, ROWS_PER_TILE)])
    if with_counts:
        pltpu.sync_copy(cnt_sh.at[pl.ds(row0, ROWS_PER_TILE)],
                        cnt_out.at[c, pl.ds(row0, ROWS_PER_TILE)])


def _aggregate(src, dst, feat, zero2, zero1, one, with_counts):
    mesh = plsc.VectorSubcoreMesh(core_axis_name="c", subcore_axis_name="s")
    f = pl.kernel(
        functools.partial(_agg_body, with_counts=with_counts),
        out_type=[
            jax.ShapeDtypeStruct((NC, N_PAD, DH), jnp.float32),
            jax.ShapeDtypeStruct((NC, N_PAD), jnp.float32),
        ],
        mesh=mesh,
        scratch_types=[
            pltpu.VMEM((2 * NBUF, CH), jnp.int32),
            pltpu.VMEM((2 * NBUF, CH), jnp.int32),
            pltpu.VMEM((CH,), jnp.float32),
            pltpu.VMEM((NBUF, CH, DH), jnp.float32),
            pltpu.VMEM_SHARED((N_PAD, DH), jnp.float32),
            pltpu.VMEM_SHARED((N_PAD, DH), jnp.float32),
            pltpu.VMEM_SHARED((N_PAD,), jnp.float32),
            pltpu.SemaphoreType.DMA((NBUF,)),
            pltpu.SemaphoreType.DMA((NBUF,)),
            pltpu.SemaphoreType.DMA,
            pltpu.SemaphoreType.DMA((2,)),
        ],
        compiler_params=pltpu.CompilerParams(use_tc_tiling_on_sc=False),
    )
    return f(src, dst, feat, zero2, zero1, one)


def _dense1_body(a_ref, cnt_ref, x_ref, wl_ref, wr_ref, b_ref, o_ref):
    cnt = jnp.maximum(cnt_ref[...], 1.0)
    mean = jnp.concatenate([a_ref[0], a_ref[1]], axis=1) / cnt
    x = jnp.concatenate([x_ref[0], x_ref[1]], axis=1)
    out = (jnp.dot(mean, wl_ref[...], preferred_element_type=jnp.float32)
           + b_ref[...]
           + jnp.dot(x, wr_ref[...], preferred_element_type=jnp.float32))
    out = jnp.maximum(out, 0.0)
    o_ref[0] = out[:, :DH]
    o_ref[1] = out[:, DH:]


def _dense2_body(a0_ref, a1_ref, cnt_ref, x0_ref, x1_ref, wl_ref, wr_ref,
                 b_ref, o_ref):
    cnt = jnp.maximum(cnt_ref[...], 1.0)
    mean = jnp.concatenate([a0_ref[0], a1_ref[0]], axis=1) / cnt
    x = jnp.concatenate([x0_ref[0], x1_ref[0]], axis=1)
    out = (jnp.dot(mean, wl_ref[...], preferred_element_type=jnp.float32)
           + b_ref[...]
           + jnp.dot(x, wr_ref[...], preferred_element_type=jnp.float32))
    out = jnp.where(jnp.isnan(out), jnp.float32(0.0), out)
    out = jnp.where(out == jnp.inf, jnp.float32(10000.0), out)
    out = jnp.where(out == -jnp.inf, jnp.float32(-10000.0), out)
    o_ref[...] = out


_BLK1 = 1280
_BLK2 = 1000


def _dense1(acc, cnt, xs, wl_t, wr_t, b):
    return pl.pallas_call(
        _dense1_body,
        grid=(N_PAD // _BLK1,),
        in_specs=[
            pl.BlockSpec((NC, _BLK1, DH), lambda i: (0, i, 0)),
            pl.BlockSpec((_BLK1, 1), lambda i: (i, 0)),
            pl.BlockSpec((NC, _BLK1, DH), lambda i: (0, i, 0)),
            pl.BlockSpec((D, D), lambda i: (0, 0)),
            pl.BlockSpec((D, D), lambda i: (0, 0)),
            pl.BlockSpec((1, D), lambda i: (0, 0)),
        ],
        out_specs=pl.BlockSpec((NC, _BLK1, DH), lambda i: (0, i, 0)),
        out_shape=jax.ShapeDtypeStruct((NC, N_PAD, DH), jnp.float32),
    )(acc, cnt, xs, wl_t, wr_t, b)


def _dense2(acc, cnt, xs, wl_t, wr_t, b):
    return pl.pallas_call(
        _dense2_body,
        grid=(N_NODES // _BLK2,),
        in_specs=[
            pl.BlockSpec((1, _BLK2, DH), lambda i: (0, i, 0)),
            pl.BlockSpec((1, _BLK2, DH), lambda i: (1, i, 0)),
            pl.BlockSpec((_BLK2, 1), lambda i: (i, 0)),
            pl.BlockSpec((1, _BLK2, DH), lambda i: (0, i, 0)),
            pl.BlockSpec((1, _BLK2, DH), lambda i: (1, i, 0)),
            pl.BlockSpec((D, D), lambda i: (0, 0)),
            pl.BlockSpec((D, D), lambda i: (0, 0)),
            pl.BlockSpec((1, D), lambda i: (0, 0)),
        ],
        out_specs=pl.BlockSpec((_BLK2, D), lambda i: (i, 0)),
        out_shape=jax.ShapeDtypeStruct((N_NODES, D), jnp.float32),
    )(acc, acc, cnt, xs, xs, wl_t, wr_t, b)


def _split_cols_pad(x):
    xp = jnp.zeros((NC, N_PAD, DH), jnp.float32)
    return xp.at[:, :x.shape[0]].set(jnp.stack([x[:, :DH], x[:, DH:]]))


def kernel(x, edge_index, W1_l, b1_l, W1_r, W2_l, b2_l, W2_r):
    src = edge_index[0].astype(jnp.int32)
    dst = edge_index[1].astype(jnp.int32)
    pad = E_PAD - N_EDGES
    src = jnp.concatenate([src, jnp.zeros((pad,), jnp.int32)])
    dst = jnp.concatenate([dst, jnp.full((pad,), DUMMY_DST, jnp.int32)])
    src = src.reshape(NS, NGRP, NBUF, CH)
    dst = dst.reshape(NS, NGRP, NBUF, CH)

    zero2 = jnp.zeros((N_PAD, DH), jnp.float32)
    zero1 = jnp.zeros((N_PAD,), jnp.float32)
    one = jnp.ones((CH,), jnp.float32)

    xs = _split_cols_pad(x)

    acc, cnt = _aggregate(src, dst, xs, zero2, zero1, one, with_counts=True)
    cnt2 = jnp.maximum(cnt[0] + cnt[1], 1.0)[:, None]

    hs = _dense1(acc, cnt2, xs, W1_l.T, W1_r.T, b1_l[None, :])

    acc2, _ = _aggregate(src, dst, hs, zero2, zero1, one, with_counts=False)
    out = _dense2(acc2, cnt2, hs, W2_l.T, W2_r.T, b2_l[None, :])
    return out


# speedup vs baseline: 1.1383x; 1.0069x over previous
import functools

import jax
import jax.numpy as jnp
from jax import lax
from jax.experimental import pallas as pl
from jax.experimental.pallas import tpu as pltpu
from jax.experimental.pallas import tpu_sc as plsc

N_NODES = 10000
N_EDGES = 320000
D = 128

NC = 2
NS = 16
DH = D // NC

CH = 128
NBUF = 5
NGRP = 32
NCH = NBUF * NGRP
E_TILE = CH * NCH
E_PAD = E_TILE * NS
N_PAD = 10240
ROWS_PER_TILE = N_PAD // NS
DUMMY_DST = N_NODES + 8


def _agg_body(src_hbm, dst_hbm, feat_hbm, zero2_hbm, zero1_hbm, one_hbm,
              acc_out, cnt_out,
              srcb_v, dstb_v, ones_v, rows_v, feat_sh, acc_sh, cnt_sh,
              gsem, ssem, csem, isem, *, with_counts):
    c = lax.axis_index("c")
    s = lax.axis_index("s")
    row0 = s * ROWS_PER_TILE

    pltpu.sync_copy(feat_hbm.at[c, pl.ds(row0, ROWS_PER_TILE)],
                    feat_sh.at[pl.ds(row0, ROWS_PER_TILE)])
    pltpu.sync_copy(zero2_hbm.at[pl.ds(row0, ROWS_PER_TILE)],
                    acc_sh.at[pl.ds(row0, ROWS_PER_TILE)])
    if with_counts:
        pltpu.sync_copy(zero1_hbm.at[pl.ds(row0, ROWS_PER_TILE)],
                        cnt_sh.at[pl.ds(row0, ROWS_PER_TILE)])
        pltpu.sync_copy(one_hbm, ones_v)
    pltpu.sync_copy(src_hbm.at[s, 0], srcb_v.at[pl.ds(0, NBUF)])
    pltpu.sync_copy(dst_hbm.at[s, 0], dstb_v.at[pl.ds(0, NBUF)])
    pltpu.async_copy(src_hbm.at[s, 1], srcb_v.at[pl.ds(NBUF, NBUF)],
                     isem.at[1])
    pltpu.async_copy(dst_hbm.at[s, 1], dstb_v.at[pl.ds(NBUF, NBUF)],
                     isem.at[1])
    plsc.subcore_barrier()

    for b in range(NBUF):
        pltpu.async_copy(feat_sh.at[srcb_v.at[b]], rows_v.at[b], gsem.at[b])

    def half(g2, par):
        pbase = par * NBUF
        qbase = (1 - par) * NBUF
        g = 2 * g2 + par
        cnt_here = jnp.logical_or(
            jnp.logical_and(c == 0, g < NGRP // 2),
            jnp.logical_and(c == 1, g >= NGRP // 2))
        scat = []
        for b in range(NBUF):
            pltpu.make_async_copy(feat_sh.at[srcb_v.at[0]],
                                  rows_v.at[b], gsem.at[b]).wait()
            scat.append(pltpu.async_copy(
                rows_v.at[b], acc_sh.at[dstb_v.at[pbase + b]], ssem.at[b],
                add=True))
            if with_counts:
                @pl.when(cnt_here)
                def _():
                    pltpu.async_copy(ones_v, cnt_sh.at[dstb_v.at[pbase + b]],
                                     csem, add=True)
        for b in range(NBUF):
            scat[b].wait()
        if with_counts:
            @pl.when(cnt_here)
            def _():
                pltpu.make_async_copy(src_hbm.at[s, 0],
                                      srcb_v.at[pl.ds(0, NBUF)], csem).wait()

        @pl.when(g + 1 < NGRP)
        def _():
            pltpu.make_async_copy(src_hbm.at[s, 0],
                                  srcb_v.at[pl.ds(qbase, NBUF)],
                                  isem.at[1 - par]).wait()
            pltpu.make_async_copy(dst_hbm.at[s, 0],
                                  dstb_v.at[pl.ds(qbase, NBUF)],
                                  isem.at[1 - par]).wait()
            for b in range(NBUF):
                pltpu.async_copy(feat_sh.at[srcb_v.at[qbase + b]],
                                 rows_v.at[b], gsem.at[b])

        @pl.when(g + 2 < NGRP)
        def _():
            pltpu.async_copy(src_hbm.at[s, g + 2],
                             srcb_v.at[pl.ds(pbase, NBUF)], isem.at[par])
            pltpu.async_copy(dst_hbm.at[s, g + 2],
                             dstb_v.at[pl.ds(pbase, NBUF)], isem.at[par])

    def super_group(g2, carry):
        half(g2, 0)
        half(g2, 1)
        return carry

    lax.fori_loop(0, NGRP // 2, super_group, 0)
    plsc.subcore_barrier()

    pltpu.sync_copy(acc_sh.at[pl.ds(row0, ROWS_PER_TILE)],
                    acc_out.at[c, pl.ds(row0, ROWS_PER_TILE)])
    if with_counts:
        pltpu.sync_copy(cnt_sh.at[pl.ds(row0, ROWS_PER_TILE)],
                        cnt_out.at[c, pl.ds(row0, ROWS_PER_TILE)])


def _aggregate(src, dst, feat, zero2, zero1, one, with_counts):
    mesh = plsc.VectorSubcoreMesh(core_axis_name="c", subcore_axis_name="s")
    f = pl.kernel(
        functools.partial(_agg_body, with_counts=with_counts),
        out_type=[
            jax.ShapeDtypeStruct((NC, N_PAD, DH), jnp.float32),
            jax.ShapeDtypeStruct((NC, N_PAD), jnp.float32),
        ],
        mesh=mesh,
        scratch_types=[
            pltpu.VMEM((2 * NBUF, CH), jnp.int32),
            pltpu.VMEM((2 * NBUF, CH), jnp.int32),
            pltpu.VMEM((CH,), jnp.float32),
            pltpu.VMEM((NBUF, CH, DH), jnp.float32),
            pltpu.VMEM_SHARED((N_PAD, DH), jnp.float32),
            pltpu.VMEM_SHARED((N_PAD, DH), jnp.float32),
            pltpu.VMEM_SHARED((N_PAD,), jnp.float32),
            pltpu.SemaphoreType.DMA((NBUF,)),
            pltpu.SemaphoreType.DMA((NBUF,)),
            pltpu.SemaphoreType.DMA,
            pltpu.SemaphoreType.DMA((2,)),
        ],
        compiler_params=pltpu.CompilerParams(use_tc_tiling_on_sc=False),
    )
    return f(src, dst, feat, zero2, zero1, one)


def _dense1_body(a_ref, cnt_ref, x_ref, wl_ref, wr_ref, b_ref, o_ref):
    cnt = jnp.maximum(cnt_ref[...], 1.0)
    mean = jnp.concatenate([a_ref[0], a_ref[1]], axis=1) / cnt
    x = jnp.concatenate([x_ref[0], x_ref[1]], axis=1)
    out = (jnp.dot(mean, wl_ref[...], preferred_element_type=jnp.float32)
           + b_ref[...]
           + jnp.dot(x, wr_ref[...], preferred_element_type=jnp.float32))
    out = jnp.maximum(out, 0.0)
    o_ref[0] = out[:, :DH]
    o_ref[1] = out[:, DH:]


def _dense2_body(a0_ref, a1_ref, cnt_ref, x0_ref, x1_ref, wl_ref, wr_ref,
                 b_ref, o_ref):
    cnt = jnp.maximum(cnt_ref[...], 1.0)
    mean = jnp.concatenate([a0_ref[0], a1_ref[0]], axis=1) / cnt
    x = jnp.concatenate([x0_ref[0], x1_ref[0]], axis=1)
    out = (jnp.dot(mean, wl_ref[...], preferred_element_type=jnp.float32)
           + b_ref[...]
           + jnp.dot(x, wr_ref[...], preferred_element_type=jnp.float32))
    out = jnp.where(jnp.isnan(out), jnp.float32(0.0), out)
    out = jnp.where(out == jnp.inf, jnp.float32(10000.0), out)
    out = jnp.where(out == -jnp.inf, jnp.float32(-10000.0), out)
    o_ref[...] = out


_BLK1 = 1280
_BLK2 = 1000


def _dense1(acc, cnt, xs, wl_t, wr_t, b):
    return pl.pallas_call(
        _dense1_body,
        grid=(N_PAD // _BLK1,),
        in_specs=[
            pl.BlockSpec((NC, _BLK1, DH), lambda i: (0, i, 0)),
            pl.BlockSpec((_BLK1, 1), lambda i: (i, 0)),
            pl.BlockSpec((NC, _BLK1, DH), lambda i: (0, i, 0)),
            pl.BlockSpec((D, D), lambda i: (0, 0)),
            pl.BlockSpec((D, D), lambda i: (0, 0)),
            pl.BlockSpec((1, D), lambda i: (0, 0)),
        ],
        out_specs=pl.BlockSpec((NC, _BLK1, DH), lambda i: (0, i, 0)),
        out_shape=jax.ShapeDtypeStruct((NC, N_PAD, DH), jnp.float32),
    )(acc, cnt, xs, wl_t, wr_t, b)


def _dense2(acc, cnt, xs, wl_t, wr_t, b):
    return pl.pallas_call(
        _dense2_body,
        grid=(N_NODES // _BLK2,),
        in_specs=[
            pl.BlockSpec((1, _BLK2, DH), lambda i: (0, i, 0)),
            pl.BlockSpec((1, _BLK2, DH), lambda i: (1, i, 0)),
            pl.BlockSpec((_BLK2, 1), lambda i: (i, 0)),
            pl.BlockSpec((1, _BLK2, DH), lambda i: (0, i, 0)),
            pl.BlockSpec((1, _BLK2, DH), lambda i: (1, i, 0)),
            pl.BlockSpec((D, D), lambda i: (0, 0)),
            pl.BlockSpec((D, D), lambda i: (0, 0)),
            pl.BlockSpec((1, D), lambda i: (0, 0)),
        ],
        out_specs=pl.BlockSpec((_BLK2, D), lambda i: (i, 0)),
        out_shape=jax.ShapeDtypeStruct((N_NODES, D), jnp.float32),
    )(acc, acc, cnt, xs, xs, wl_t, wr_t, b)


def _split_cols_pad(x):
    xp = jnp.zeros((NC, N_PAD, DH), jnp.float32)
    return xp.at[:, :x.shape[0]].set(jnp.stack([x[:, :DH], x[:, DH:]]))


def kernel(x, edge_index, W1_l, b1_l, W1_r, W2_l, b2_l, W2_r):
    src = edge_index[0].astype(jnp.int32)
    dst = edge_index[1].astype(jnp.int32)
    pad = E_PAD - N_EDGES
    src = jnp.concatenate([src, jnp.zeros((pad,), jnp.int32)])
    dst = jnp.concatenate([dst, jnp.full((pad,), DUMMY_DST, jnp.int32)])
    src = src.reshape(NS, NGRP, NBUF, CH)
    dst = dst.reshape(NS, NGRP, NBUF, CH)

    zero2 = jnp.zeros((N_PAD, DH), jnp.float32)
    zero1 = jnp.zeros((N_PAD,), jnp.float32)
    one = jnp.ones((CH,), jnp.float32)

    xs = _split_cols_pad(x)

    acc, cnt = _aggregate(src, dst, xs, zero2, zero1, one, with_counts=True)
    cnt2 = jnp.maximum(cnt[0] + cnt[1], 1.0)[:, None]

    hs = _dense1(acc, cnt2, xs, W1_l.T, W1_r.T, b1_l[None, :])

    acc2, _ = _aggregate(src, dst, hs, zero2, zero1, one, with_counts=False)
    out = _dense2(acc2, cnt2, hs, W2_l.T, W2_r.T, b2_l[None, :])
    return out
